# Initial kernel scaffold; baseline (speedup 1.0000x reference)
import jax
import jax.numpy as jnp
from jax.experimental import pallas as pl


def kernel(x, edge_index, Wq1, bq1, Wk1, bk1, Wv1, bv1, Ws1, bs1, g1, be1, Wq2, bq2, Wk2, bk2, Wv2, bv2, Ws2, bs2, g2, be2):
    raise NotImplementedError("write your pallas kernel here")


# speedup vs baseline: 5.9327x; 5.9327x over previous
import functools
import math

import jax
import jax.numpy as jnp
from jax import lax
from jax.experimental import pallas as pl
from jax.experimental.pallas import tpu as pltpu
from jax.experimental.pallas import tpu_sc as plsc

_N = 10000
_E = 320000
_D = 128
_AW = _D + 16
_B = 40
_NTILES = 32
_EPT = _E // _NTILES
_NCHUNK = _EPT // _B
_RPT = _N // 16
_ROWBLK = 1000
_GRID = _N // _ROWBLK


def _proj_body(x_ref, w_ref, b_ref, q_ref, kv_ref, sk_ref):
    y = lax.dot_general(x_ref[...], w_ref[...], (((1,), (1,)), ((), ())),
                        preferred_element_type=jnp.float32,
                        precision=lax.Precision.HIGHEST)
    y = y + b_ref[...]
    q_ref[...] = y[:, :_D]
    kv_ref[...] = y[:, _D:3 * _D]
    sk_ref[...] = y[:, 3 * _D:]


def _proj(x, w, b):
    return pl.pallas_call(
        _proj_body,
        grid=(_GRID,),
        in_specs=[
            pl.BlockSpec((_ROWBLK, _D), lambda i: (i, 0)),
            pl.BlockSpec((4 * _D, _D), lambda i: (0, 0)),
            pl.BlockSpec((1, 4 * _D), lambda i: (0, 0)),
        ],
        out_specs=[
            pl.BlockSpec((_ROWBLK, _D), lambda i: (i, 0)),
            pl.BlockSpec((_ROWBLK, 2 * _D), lambda i: (i, 0)),
            pl.BlockSpec((_ROWBLK, _D), lambda i: (i, 0)),
        ],
        out_shape=[
            jax.ShapeDtypeStruct((_N, _D), jnp.float32),
            jax.ShapeDtypeStruct((_N, 2 * _D), jnp.float32),
            jax.ShapeDtypeStruct((_N, _D), jnp.float32),
        ],
    )(x, w, b)


def _norm_block(acc, sk, g, be):
    s = acc[0] + acc[1]
    o = s[:, :_D] / (s[:, _D:_D + 1] + 1e-16) + sk
    mu = jnp.mean(o, axis=1, keepdims=True)
    var = jnp.mean((o - mu) ** 2, axis=1, keepdims=True)
    return (o - mu) / jnp.sqrt(var + 1e-5) * g + be


def _comb_proj_body(acc_ref, sk_ref, g_ref, be_ref, w_ref, b_ref,
                    q_ref, kv_ref, sk2_ref):
    h = _norm_block(acc_ref[...], sk_ref[...], g_ref[...], be_ref[...])
    y = lax.dot_general(h, w_ref[...], (((1,), (1,)), ((), ())),
                        preferred_element_type=jnp.float32,
                        precision=lax.Precision.HIGHEST)
    y = y + b_ref[...]
    q_ref[...] = y[:, :_D]
    kv_ref[...] = y[:, _D:3 * _D]
    sk2_ref[...] = y[:, 3 * _D:]


def _comb_proj(acc, sk, g, be, w, b):
    return pl.pallas_call(
        _comb_proj_body,
        grid=(_GRID,),
        in_specs=[
            pl.BlockSpec((2, _ROWBLK, _AW), lambda i: (0, i, 0)),
            pl.BlockSpec((_ROWBLK, _D), lambda i: (i, 0)),
            pl.BlockSpec((1, _D), lambda i: (0, 0)),
            pl.BlockSpec((1, _D), lambda i: (0, 0)),
            pl.BlockSpec((4 * _D, _D), lambda i: (0, 0)),
            pl.BlockSpec((1, 4 * _D), lambda i: (0, 0)),
        ],
        out_specs=[
            pl.BlockSpec((_ROWBLK, _D), lambda i: (i, 0)),
            pl.BlockSpec((_ROWBLK, 2 * _D), lambda i: (i, 0)),
            pl.BlockSpec((_ROWBLK, _D), lambda i: (i, 0)),
        ],
        out_shape=[
            jax.ShapeDtypeStruct((_N, _D), jnp.float32),
            jax.ShapeDtypeStruct((_N, 2 * _D), jnp.float32),
            jax.ShapeDtypeStruct((_N, _D), jnp.float32),
        ],
    )(acc, sk, g, be, w, b)


def _comb_final_body(acc_ref, sk_ref, g_ref, be_ref, h_ref):
    h_ref[...] = _norm_block(acc_ref[...], sk_ref[...], g_ref[...], be_ref[...])


def _comb_final(acc, sk, g, be):
    return pl.pallas_call(
        _comb_final_body,
        grid=(_GRID,),
        in_specs=[
            pl.BlockSpec((2, _ROWBLK, _AW), lambda i: (0, i, 0)),
            pl.BlockSpec((_ROWBLK, _D), lambda i: (i, 0)),
            pl.BlockSpec((1, _D), lambda i: (0, 0)),
            pl.BlockSpec((1, _D), lambda i: (0, 0)),
        ],
        out_specs=pl.BlockSpec((_ROWBLK, _D), lambda i: (i, 0)),
        out_shape=jax.ShapeDtypeStruct((_N, _D), jnp.float32),
    )(acc, sk, g, be)


_INV_SQRT_C = 1.0 / math.sqrt(_D)


@functools.partial(
    pl.kernel,
    mesh=plsc.VectorSubcoreMesh(core_axis_name="c", subcore_axis_name="s"),
    compiler_params=pltpu.CompilerParams(use_tc_tiling_on_sc=False,
                                         needs_layout_passes=False),
    out_type=jax.ShapeDtypeStruct((2, _N, _AW), jnp.float32),
    scratch_types=[
        pltpu.VMEM((_B,), jnp.int32),
        pltpu.VMEM((_B,), jnp.int32),
        pltpu.VMEM((_B, _D), jnp.float32),
        pltpu.VMEM((_B, 2 * _D), jnp.float32),
        pltpu.VMEM((_B, _AW), jnp.float32),
        pltpu.VMEM_SHARED((_N, _AW), jnp.float32),
        pltpu.SemaphoreType.DMA,
        pltpu.SemaphoreType.DMA,
    ],
)
def _edge_kernel(q_hbm, kv_hbm, src_hbm, dst_hbm, out_hbm,
                 didx, sidx, qr, kvr, stage, acc, sem_q, sem_kv):
    cid = lax.axis_index("c")
    sid = lax.axis_index("s")

    def _zrow(i, carry):
        for c in range(_AW // 16):
            stage[i, pl.ds(16 * c, 16)] = jnp.zeros((16,), jnp.float32)
        return carry
    lax.fori_loop(0, _B, _zrow, 0)
    row0 = sid * _RPT
    nfull = _RPT // _B
    rem = _RPT - nfull * _B
    for r in range(nfull):
        pltpu.sync_copy(stage, acc.at[pl.ds(row0 + r * _B, _B)])
    pltpu.sync_copy(stage.at[pl.ds(0, rem)],
                    acc.at[pl.ds(row0 + nfull * _B, rem)])
    plsc.subcore_barrier()

    base0 = (cid * 16 + sid) * _EPT

    def _chunk(it, carry):
        base = base0 + it * _B
        pltpu.sync_copy(dst_hbm.at[pl.ds(base, _B)], didx)
        pltpu.sync_copy(src_hbm.at[pl.ds(base, _B)], sidx)
        cq = pltpu.async_copy(q_hbm.at[didx], qr, sem_q)
        ckv = pltpu.async_copy(kv_hbm.at[sidx], kvr, sem_kv)
        cq.wait()
        ckv.wait()

        def _edge(e, ecarry):
            p = qr[e, pl.ds(0, 16)] * kvr[e, pl.ds(0, 16)]
            for c in range(1, 8):
                p = p + qr[e, pl.ds(16 * c, 16)] * kvr[e, pl.ds(16 * c, 16)]
            a = jnp.sum(p) * _INV_SQRT_C
            s = jnp.exp(jnp.full((16,), a, jnp.float32))
            for c in range(8):
                stage[e, pl.ds(16 * c, 16)] = s * kvr[e, pl.ds(_D + 16 * c, 16)]
            stage[e, pl.ds(_D, 16)] = s
            return ecarry
        lax.fori_loop(0, _B, _edge, 0)

        pltpu.sync_copy(stage, acc.at[didx], add=True)
        return carry

    lax.fori_loop(0, _NCHUNK, _chunk, 0)
    plsc.subcore_barrier()

    pltpu.sync_copy(acc.at[pl.ds(sid * _RPT, _RPT)],
                    out_hbm.at[cid, pl.ds(sid * _RPT, _RPT)])


def kernel(x, edge_index, Wq1, bq1, Wk1, bk1, Wv1, bv1, Ws1, bs1, g1, be1,
           Wq2, bq2, Wk2, bk2, Wv2, bv2, Ws2, bs2, g2, be2):
    w1 = jnp.concatenate([Wq1, Wk1, Wv1, Ws1], axis=0)
    b1 = jnp.concatenate([bq1, bk1, bv1, bs1])[None, :]
    w2 = jnp.concatenate([Wq2, Wk2, Wv2, Ws2], axis=0)
    b2 = jnp.concatenate([bq2, bk2, bv2, bs2])[None, :]
    src = edge_index[0]
    dst = edge_index[1]

    q1, kv1, sk1 = _proj(x, w1, b1)
    acc1 = _edge_kernel(q1, kv1, src, dst)
    q2, kv2, sk2 = _comb_proj(acc1, sk1, g1[None, :], be1[None, :], w2, b2)
    acc2 = _edge_kernel(q2, kv2, src, dst)
    return _comb_final(acc2, sk2, g2[None, :], be2[None, :])


# speedup vs baseline: 9.2118x; 1.5527x over previous
import functools
import math

import jax
import jax.numpy as jnp
from jax import lax
from jax.experimental import pallas as pl
from jax.experimental.pallas import tpu as pltpu
from jax.experimental.pallas import tpu_sc as plsc

_N = 10000
_E = 320000
_D = 128
_AW = _D + 16
_B = 40
_NTILES = 32
_EPT = _E // _NTILES
_NCHUNK = _EPT // _B
_RPT = _N // 16
_ROWBLK = 1000
_GRID = _N // _ROWBLK


def _proj_body(x_ref, w_ref, b_ref, q_ref, kv_ref, sk_ref):
    y = lax.dot_general(x_ref[...], w_ref[...], (((1,), (1,)), ((), ())),
                        preferred_element_type=jnp.float32,
                        precision=lax.Precision.HIGHEST)
    y = y + b_ref[...]
    q_ref[...] = y[:, :_D]
    kv_ref[...] = y[:, _D:3 * _D]
    sk_ref[...] = y[:, 3 * _D:]


def _proj(x, w, b):
    return pl.pallas_call(
        _proj_body,
        grid=(_GRID,),
        in_specs=[
            pl.BlockSpec((_ROWBLK, _D), lambda i: (i, 0)),
            pl.BlockSpec((4 * _D, _D), lambda i: (0, 0)),
            pl.BlockSpec((1, 4 * _D), lambda i: (0, 0)),
        ],
        out_specs=[
            pl.BlockSpec((_ROWBLK, _D), lambda i: (i, 0)),
            pl.BlockSpec((_ROWBLK, 2 * _D), lambda i: (i, 0)),
            pl.BlockSpec((_ROWBLK, _D), lambda i: (i, 0)),
        ],
        out_shape=[
            jax.ShapeDtypeStruct((_N, _D), jnp.float32),
            jax.ShapeDtypeStruct((_N, 2 * _D), jnp.float32),
            jax.ShapeDtypeStruct((_N, _D), jnp.float32),
        ],
    )(x, w, b)


def _norm_block(acc, sk, g, be):
    s = acc[0] + acc[1]
    o = s[:, :_D] / (s[:, _D:_D + 1] + 1e-16) + sk
    mu = jnp.mean(o, axis=1, keepdims=True)
    var = jnp.mean((o - mu) ** 2, axis=1, keepdims=True)
    return (o - mu) / jnp.sqrt(var + 1e-5) * g + be


def _comb_proj_body(acc_ref, sk_ref, g_ref, be_ref, w_ref, b_ref,
                    q_ref, kv_ref, sk2_ref):
    h = _norm_block(acc_ref[...], sk_ref[...], g_ref[...], be_ref[...])
    y = lax.dot_general(h, w_ref[...], (((1,), (1,)), ((), ())),
                        preferred_element_type=jnp.float32,
                        precision=lax.Precision.HIGHEST)
    y = y + b_ref[...]
    q_ref[...] = y[:, :_D]
    kv_ref[...] = y[:, _D:3 * _D]
    sk2_ref[...] = y[:, 3 * _D:]


def _comb_proj(acc, sk, g, be, w, b):
    return pl.pallas_call(
        _comb_proj_body,
        grid=(_GRID,),
        in_specs=[
            pl.BlockSpec((2, _ROWBLK, _AW), lambda i: (0, i, 0)),
            pl.BlockSpec((_ROWBLK, _D), lambda i: (i, 0)),
            pl.BlockSpec((1, _D), lambda i: (0, 0)),
            pl.BlockSpec((1, _D), lambda i: (0, 0)),
            pl.BlockSpec((4 * _D, _D), lambda i: (0, 0)),
            pl.BlockSpec((1, 4 * _D), lambda i: (0, 0)),
        ],
        out_specs=[
            pl.BlockSpec((_ROWBLK, _D), lambda i: (i, 0)),
            pl.BlockSpec((_ROWBLK, 2 * _D), lambda i: (i, 0)),
            pl.BlockSpec((_ROWBLK, _D), lambda i: (i, 0)),
        ],
        out_shape=[
            jax.ShapeDtypeStruct((_N, _D), jnp.float32),
            jax.ShapeDtypeStruct((_N, 2 * _D), jnp.float32),
            jax.ShapeDtypeStruct((_N, _D), jnp.float32),
        ],
    )(acc, sk, g, be, w, b)


def _comb_final_body(acc_ref, sk_ref, g_ref, be_ref, h_ref):
    h_ref[...] = _norm_block(acc_ref[...], sk_ref[...], g_ref[...], be_ref[...])


def _comb_final(acc, sk, g, be):
    return pl.pallas_call(
        _comb_final_body,
        grid=(_GRID,),
        in_specs=[
            pl.BlockSpec((2, _ROWBLK, _AW), lambda i: (0, i, 0)),
            pl.BlockSpec((_ROWBLK, _D), lambda i: (i, 0)),
            pl.BlockSpec((1, _D), lambda i: (0, 0)),
            pl.BlockSpec((1, _D), lambda i: (0, 0)),
        ],
        out_specs=pl.BlockSpec((_ROWBLK, _D), lambda i: (i, 0)),
        out_shape=jax.ShapeDtypeStruct((_N, _D), jnp.float32),
    )(acc, sk, g, be)


_INV_SQRT_C = 1.0 / math.sqrt(_D)


_SEG = 1000
_CPS = _SEG // _B
_NSEG = _EPT // _SEG


@functools.partial(
    pl.kernel,
    mesh=plsc.VectorSubcoreMesh(core_axis_name="c", subcore_axis_name="s"),
    compiler_params=pltpu.CompilerParams(use_tc_tiling_on_sc=False,
                                         needs_layout_passes=False),
    out_type=jax.ShapeDtypeStruct((2, _N, _AW), jnp.float32),
    scratch_types=[
        pltpu.VMEM((2, _CPS, _B), jnp.int32),
        pltpu.VMEM((2, _CPS, _B), jnp.int32),
        pltpu.VMEM((2, _B, _D), jnp.float32),
        pltpu.VMEM((2, _B, 2 * _D), jnp.float32),
        pltpu.VMEM((_B, _AW), jnp.float32),
        pltpu.VMEM_SHARED((_N, _AW), jnp.float32),
        pltpu.SemaphoreType.DMA,
        pltpu.SemaphoreType.DMA,
        pltpu.SemaphoreType.DMA,
    ],
)
def _edge_kernel(q_hbm, kv_hbm, src_hbm, dst_hbm, out_hbm,
                 didx, sidx, qr2, kvr2, stage, acc, sem_q, sem_kv, sem_i):
    cid = lax.axis_index("c")
    sid = lax.axis_index("s")

    def _zrow(i, carry):
        for c in range(_AW // 16):
            stage[i, pl.ds(16 * c, 16)] = jnp.zeros((16,), jnp.float32)
        return carry
    lax.fori_loop(0, _B, _zrow, 0)
    row0 = sid * _RPT
    nfull = _RPT // _B
    rem = _RPT - nfull * _B
    for r in range(nfull):
        pltpu.sync_copy(stage, acc.at[pl.ds(row0 + r * _B, _B)])
    pltpu.sync_copy(stage.at[pl.ds(0, rem)],
                    acc.at[pl.ds(row0 + nfull * _B, rem)])
    plsc.subcore_barrier()

    cbase0 = (cid * 16 + sid) * (_EPT // _B)

    def _fire_gather(it):
        j = lax.rem(it, _CPS)
        sp = lax.rem(lax.div(it, _CPS), 2)
        p = lax.rem(it, 2)
        pltpu.async_copy(q_hbm.at[didx.at[sp, j]], qr2.at[p], sem_q)
        pltpu.async_copy(kv_hbm.at[sidx.at[sp, j]], kvr2.at[p], sem_kv)

    def _fire_ids(seg):
        sp = lax.rem(seg, 2)
        pltpu.async_copy(dst_hbm.at[pl.ds(cbase0 + seg * _CPS, _CPS)],
                         didx.at[sp], sem_i)
        pltpu.async_copy(src_hbm.at[pl.ds(cbase0 + seg * _CPS, _CPS)],
                         sidx.at[sp], sem_i)

    def _wait_ids():
        pltpu.make_async_copy(dst_hbm.at[pl.ds(cbase0, _CPS)],
                              didx.at[0], sem_i).wait()
        pltpu.make_async_copy(src_hbm.at[pl.ds(cbase0, _CPS)],
                              sidx.at[0], sem_i).wait()

    _fire_ids(0)
    _wait_ids()
    _fire_gather(0)

    def _chunk(it, carry):
        j = lax.rem(it, _CPS)
        seg = lax.div(it, _CPS)
        sp = lax.rem(seg, 2)
        p = lax.rem(it, 2)

        pltpu.make_async_copy(q_hbm.at[didx.at[0, 0]], qr2.at[p],
                              sem_q).wait()
        pltpu.make_async_copy(kv_hbm.at[sidx.at[0, 0]], kvr2.at[p],
                              sem_kv).wait()

        @pl.when(jnp.logical_and(j == 0, seg < _NSEG - 1))
        def _():
            _fire_ids(seg + 1)

        @pl.when(jnp.logical_and(j == _CPS - 1, seg < _NSEG - 1))
        def _():
            _wait_ids()

        @pl.when(it < _NCHUNK - 1)
        def _():
            _fire_gather(it + 1)

        def _edge(e, ecarry):
            pr = qr2[p, e, pl.ds(0, 16)] * kvr2[p, e, pl.ds(0, 16)]
            for c in range(1, 8):
                pr = pr + (qr2[p, e, pl.ds(16 * c, 16)]
                           * kvr2[p, e, pl.ds(16 * c, 16)])
            a = jnp.sum(pr) * _INV_SQRT_C
            s = jnp.exp(jnp.full((16,), a, jnp.float32))
            for c in range(8):
                stage[e, pl.ds(16 * c, 16)] = (
                    s * kvr2[p, e, pl.ds(_D + 16 * c, 16)])
            stage[e, pl.ds(_D, 16)] = s
            return ecarry
        lax.fori_loop(0, _B, _edge, 0, unroll=4)

        pltpu.sync_copy(stage, acc.at[didx.at[sp, j]], add=True)
        return carry

    lax.fori_loop(0, _NCHUNK, _chunk, 0)
    plsc.subcore_barrier()

    pltpu.sync_copy(acc.at[pl.ds(sid * _RPT, _RPT)],
                    out_hbm.at[cid, pl.ds(sid * _RPT, _RPT)])


def kernel(x, edge_index, Wq1, bq1, Wk1, bk1, Wv1, bv1, Ws1, bs1, g1, be1,
           Wq2, bq2, Wk2, bk2, Wv2, bv2, Ws2, bs2, g2, be2):
    w1 = jnp.concatenate([Wq1, Wk1, Wv1, Ws1], axis=0)
    b1 = jnp.concatenate([bq1, bk1, bv1, bs1])[None, :]
    w2 = jnp.concatenate([Wq2, Wk2, Wv2, Ws2], axis=0)
    b2 = jnp.concatenate([bq2, bk2, bv2, bs2])[None, :]
    src = edge_index[0].reshape(_E // _B, _B)
    dst = edge_index[1].reshape(_E // _B, _B)

    q1, kv1, sk1 = _proj(x, w1, b1)
    acc1 = _edge_kernel(q1, kv1, src, dst)
    q2, kv2, sk2 = _comb_proj(acc1, sk1, g1[None, :], be1[None, :], w2, b2)
    acc2 = _edge_kernel(q2, kv2, src, dst)
    return _comb_final(acc2, sk2, g2[None, :], be2[None, :])


# speedup vs baseline: 9.2645x; 1.0057x over previous
import functools
import math

import jax
import jax.numpy as jnp
from jax import lax
from jax.experimental import pallas as pl
from jax.experimental.pallas import tpu as pltpu
from jax.experimental.pallas import tpu_sc as plsc

_N = 10000
_E = 320000
_D = 128
_AW = _D + 16
_B = 40
_NTILES = 32
_EPT = _E // _NTILES
_NCHUNK = _EPT // _B
_RPT = _N // 16
_ROWBLK = 1000
_GRID = _N // _ROWBLK


def _proj_body(x_ref, w_ref, b_ref, q_ref, kv_ref, sk_ref):
    y = lax.dot_general(x_ref[...], w_ref[...], (((1,), (1,)), ((), ())),
                        preferred_element_type=jnp.float32,
                        precision=lax.Precision.HIGHEST)
    y = y + b_ref[...]
    q_ref[...] = y[:, :_D]
    kv_ref[...] = y[:, _D:3 * _D]
    sk_ref[...] = y[:, 3 * _D:]


def _proj(x, w, b):
    return pl.pallas_call(
        _proj_body,
        grid=(_GRID,),
        in_specs=[
            pl.BlockSpec((_ROWBLK, _D), lambda i: (i, 0)),
            pl.BlockSpec((4 * _D, _D), lambda i: (0, 0)),
            pl.BlockSpec((1, 4 * _D), lambda i: (0, 0)),
        ],
        out_specs=[
            pl.BlockSpec((_ROWBLK, _D), lambda i: (i, 0)),
            pl.BlockSpec((_ROWBLK, 2 * _D), lambda i: (i, 0)),
            pl.BlockSpec((_ROWBLK, _D), lambda i: (i, 0)),
        ],
        out_shape=[
            jax.ShapeDtypeStruct((_N, _D), jnp.float32),
            jax.ShapeDtypeStruct((_N, 2 * _D), jnp.float32),
            jax.ShapeDtypeStruct((_N, _D), jnp.float32),
        ],
    )(x, w, b)


def _norm_block(acc, sk, g, be):
    s = acc[0] + acc[1]
    o = s[:, :_D] / (s[:, _D:_D + 1] + 1e-16) + sk
    mu = jnp.mean(o, axis=1, keepdims=True)
    var = jnp.mean((o - mu) ** 2, axis=1, keepdims=True)
    return (o - mu) / jnp.sqrt(var + 1e-5) * g + be


def _comb_proj_body(acc_ref, sk_ref, g_ref, be_ref, w_ref, b_ref,
                    q_ref, kv_ref, sk2_ref):
    h = _norm_block(acc_ref[...], sk_ref[...], g_ref[...], be_ref[...])
    y = lax.dot_general(h, w_ref[...], (((1,), (1,)), ((), ())),
                        preferred_element_type=jnp.float32,
                        precision=lax.Precision.HIGHEST)
    y = y + b_ref[...]
    q_ref[...] = y[:, :_D]
    kv_ref[...] = y[:, _D:3 * _D]
    sk2_ref[...] = y[:, 3 * _D:]


def _comb_proj(acc, sk, g, be, w, b):
    return pl.pallas_call(
        _comb_proj_body,
        grid=(_GRID,),
        in_specs=[
            pl.BlockSpec((2, _ROWBLK, _AW), lambda i: (0, i, 0)),
            pl.BlockSpec((_ROWBLK, _D), lambda i: (i, 0)),
            pl.BlockSpec((1, _D), lambda i: (0, 0)),
            pl.BlockSpec((1, _D), lambda i: (0, 0)),
            pl.BlockSpec((4 * _D, _D), lambda i: (0, 0)),
            pl.BlockSpec((1, 4 * _D), lambda i: (0, 0)),
        ],
        out_specs=[
            pl.BlockSpec((_ROWBLK, _D), lambda i: (i, 0)),
            pl.BlockSpec((_ROWBLK, 2 * _D), lambda i: (i, 0)),
            pl.BlockSpec((_ROWBLK, _D), lambda i: (i, 0)),
        ],
        out_shape=[
            jax.ShapeDtypeStruct((_N, _D), jnp.float32),
            jax.ShapeDtypeStruct((_N, 2 * _D), jnp.float32),
            jax.ShapeDtypeStruct((_N, _D), jnp.float32),
        ],
    )(acc, sk, g, be, w, b)


def _comb_final_body(acc_ref, sk_ref, g_ref, be_ref, h_ref):
    h_ref[...] = _norm_block(acc_ref[...], sk_ref[...], g_ref[...], be_ref[...])


def _comb_final(acc, sk, g, be):
    return pl.pallas_call(
        _comb_final_body,
        grid=(_GRID,),
        in_specs=[
            pl.BlockSpec((2, _ROWBLK, _AW), lambda i: (0, i, 0)),
            pl.BlockSpec((_ROWBLK, _D), lambda i: (i, 0)),
            pl.BlockSpec((1, _D), lambda i: (0, 0)),
            pl.BlockSpec((1, _D), lambda i: (0, 0)),
        ],
        out_specs=pl.BlockSpec((_ROWBLK, _D), lambda i: (i, 0)),
        out_shape=jax.ShapeDtypeStruct((_N, _D), jnp.float32),
    )(acc, sk, g, be)


_INV_SQRT_C = 1.0 / math.sqrt(_D)


_CPS = 10
_NSEG = _NCHUNK // _CPS
_HB = _B // 2


@functools.partial(
    pl.kernel,
    mesh=plsc.VectorSubcoreMesh(core_axis_name="c", subcore_axis_name="s"),
    compiler_params=pltpu.CompilerParams(use_tc_tiling_on_sc=False,
                                         needs_layout_passes=False),
    out_type=jax.ShapeDtypeStruct((2, _N, _AW), jnp.float32),
    scratch_types=[
        pltpu.VMEM((2, _CPS, _B), jnp.int32),
        pltpu.VMEM((2, _CPS, _B), jnp.int32),
        pltpu.VMEM((2, _CPS, 2, _HB), jnp.int32),
        pltpu.VMEM((2, _B, _D), jnp.float32),
        pltpu.VMEM((2, _B, 2 * _D), jnp.float32),
        pltpu.VMEM((2, _HB, _AW), jnp.float32),
        pltpu.VMEM_SHARED((_N, _AW), jnp.float32),
        pltpu.SemaphoreType.DMA,
        pltpu.SemaphoreType.DMA,
        pltpu.SemaphoreType.DMA,
        pltpu.SemaphoreType.DMA,
    ],
)
def _edge_kernel(q_hbm, kv_hbm, src_hbm, dst_hbm, dsth_hbm, out_hbm,
                 didx, sidx, didxh, qr2, kvr2, stage, acc,
                 sem_q, sem_kv, sem_i, sem_s):
    cid = lax.axis_index("c")
    sid = lax.axis_index("s")

    def _zrow(i, carry):
        for hp in range(2):
            for c in range(_AW // 16):
                stage[hp, i, pl.ds(16 * c, 16)] = jnp.zeros((16,), jnp.float32)
        return carry
    lax.fori_loop(0, _HB, _zrow, 0)
    row0 = sid * _RPT
    nfull = _RPT // _HB
    rem = _RPT - nfull * _HB
    for r in range(nfull):
        pltpu.sync_copy(stage.at[0], acc.at[pl.ds(row0 + r * _HB, _HB)])
    pltpu.sync_copy(stage.at[0, pl.ds(0, rem)],
                    acc.at[pl.ds(row0 + nfull * _HB, rem)])
    plsc.subcore_barrier()

    cbase0 = (cid * 16 + sid) * (_EPT // _B)

    def _fire_gather(it):
        j = lax.rem(it, _CPS)
        sp = lax.rem(lax.div(it, _CPS), 2)
        p = lax.rem(it, 2)
        pltpu.async_copy(q_hbm.at[didx.at[sp, j]], qr2.at[p], sem_q)
        pltpu.async_copy(kv_hbm.at[sidx.at[sp, j]], kvr2.at[p], sem_kv)

    def _fire_ids(seg):
        sp = lax.rem(seg, 2)
        pltpu.async_copy(dst_hbm.at[pl.ds(cbase0 + seg * _CPS, _CPS)],
                         didx.at[sp], sem_i)
        pltpu.async_copy(src_hbm.at[pl.ds(cbase0 + seg * _CPS, _CPS)],
                         sidx.at[sp], sem_i)
        pltpu.async_copy(dsth_hbm.at[pl.ds(cbase0 + seg * _CPS, _CPS)],
                         didxh.at[sp], sem_i)

    def _wait_ids():
        pltpu.make_async_copy(dst_hbm.at[pl.ds(cbase0, _CPS)],
                              didx.at[0], sem_i).wait()
        pltpu.make_async_copy(src_hbm.at[pl.ds(cbase0, _CPS)],
                              sidx.at[0], sem_i).wait()
        pltpu.make_async_copy(dsth_hbm.at[pl.ds(cbase0, _CPS)],
                              didxh.at[0], sem_i).wait()

    _fire_ids(0)
    _wait_ids()
    _fire_gather(0)

    def _chunk(it, carry):
        j = lax.rem(it, _CPS)
        seg = lax.div(it, _CPS)
        sp = lax.rem(seg, 2)
        p = lax.rem(it, 2)

        pltpu.make_async_copy(q_hbm.at[didx.at[0, 0]], qr2.at[p],
                              sem_q).wait()
        pltpu.make_async_copy(kv_hbm.at[sidx.at[0, 0]], kvr2.at[p],
                              sem_kv).wait()

        @pl.when(jnp.logical_and(j == 0, seg < _NSEG - 1))
        def _():
            _fire_ids(seg + 1)

        @pl.when(jnp.logical_and(j == _CPS - 1, seg < _NSEG - 1))
        def _():
            _wait_ids()

        @pl.when(it < _NCHUNK - 1)
        def _():
            _fire_gather(it + 1)

        @pl.when(it > 0)
        def _():
            for _h in range(2):
                pltpu.make_async_copy(stage.at[0],
                                      acc.at[didxh.at[0, 0, 0]],
                                      sem_s).wait()

        for h in range(2):
            def _edge(e, ecarry, h=h):
                ge = h * _HB + e
                pr = qr2[p, ge, pl.ds(0, 16)] * kvr2[p, ge, pl.ds(0, 16)]
                for c in range(1, 8):
                    pr = pr + (qr2[p, ge, pl.ds(16 * c, 16)]
                               * kvr2[p, ge, pl.ds(16 * c, 16)])
                a = jnp.sum(pr) * _INV_SQRT_C
                s = jnp.exp(jnp.full((16,), a, jnp.float32))
                for c in range(8):
                    stage[h, e, pl.ds(16 * c, 16)] = (
                        s * kvr2[p, ge, pl.ds(_D + 16 * c, 16)])
                stage[h, e, pl.ds(_D, 16)] = s
                return ecarry
            lax.fori_loop(0, _HB, _edge, 0, unroll=4)
            pltpu.async_copy(stage.at[h], acc.at[didxh.at[sp, j, h]],
                             sem_s, add=True)
        return carry

    lax.fori_loop(0, _NCHUNK, _chunk, 0)
    for _h in range(2):
        pltpu.make_async_copy(stage.at[0], acc.at[didxh.at[0, 0, 0]],
                              sem_s).wait()
    plsc.subcore_barrier()

    pltpu.sync_copy(acc.at[pl.ds(sid * _RPT, _RPT)],
                    out_hbm.at[cid, pl.ds(sid * _RPT, _RPT)])


def kernel(x, edge_index, Wq1, bq1, Wk1, bk1, Wv1, bv1, Ws1, bs1, g1, be1,
           Wq2, bq2, Wk2, bk2, Wv2, bv2, Ws2, bs2, g2, be2):
    w1 = jnp.concatenate([Wq1, Wk1, Wv1, Ws1], axis=0)
    b1 = jnp.concatenate([bq1, bk1, bv1, bs1])[None, :]
    w2 = jnp.concatenate([Wq2, Wk2, Wv2, Ws2], axis=0)
    b2 = jnp.concatenate([bq2, bk2, bv2, bs2])[None, :]
    src = edge_index[0].reshape(_E // _B, _B)
    dst = edge_index[1].reshape(_E // _B, _B)
    dsth = edge_index[1].reshape(_E // _B, 2, _HB)

    q1, kv1, sk1 = _proj(x, w1, b1)
    acc1 = _edge_kernel(q1, kv1, src, dst, dsth)
    q2, kv2, sk2 = _comb_proj(acc1, sk1, g1[None, :], be1[None, :], w2, b2)
    acc2 = _edge_kernel(q2, kv2, src, dst, dsth)
    return _comb_final(acc2, sk2, g2[None, :], be2[None, :])


# speedup vs baseline: 18.5142x; 1.9984x over previous
import functools
import math

import jax
import jax.numpy as jnp
from jax import lax
from jax.experimental import pallas as pl
from jax.experimental.pallas import tpu as pltpu
from jax.experimental.pallas import tpu_sc as plsc

_N = 10000
_E = 320000
_D = 128
_AW = _D + 16
_B = 40
_NTILES = 32
_EPT = _E // _NTILES
_NCHUNK = _EPT // _B
_RPT = _N // 16
_ROWBLK = 1000
_GRID = _N // _ROWBLK


def _proj_body(x_ref, w_ref, b_ref, q_ref, kv_ref, sk_ref):
    y = lax.dot_general(x_ref[...], w_ref[...], (((1,), (1,)), ((), ())),
                        preferred_element_type=jnp.float32,
                        precision=lax.Precision.HIGHEST)
    y = y + b_ref[...]
    q_ref[...] = y[:, :_D]
    kv_ref[...] = y[:, _D:3 * _D]
    sk_ref[...] = y[:, 3 * _D:]


def _proj(x, w, b):
    return pl.pallas_call(
        _proj_body,
        grid=(_GRID,),
        in_specs=[
            pl.BlockSpec((_ROWBLK, _D), lambda i: (i, 0)),
            pl.BlockSpec((4 * _D, _D), lambda i: (0, 0)),
            pl.BlockSpec((1, 4 * _D), lambda i: (0, 0)),
        ],
        out_specs=[
            pl.BlockSpec((_ROWBLK, _D), lambda i: (i, 0)),
            pl.BlockSpec((_ROWBLK, 2 * _D), lambda i: (i, 0)),
            pl.BlockSpec((_ROWBLK, _D), lambda i: (i, 0)),
        ],
        out_shape=[
            jax.ShapeDtypeStruct((_N, _D), jnp.float32),
            jax.ShapeDtypeStruct((_N, 2 * _D), jnp.float32),
            jax.ShapeDtypeStruct((_N, _D), jnp.float32),
        ],
    )(x, w, b)


def _norm_block(acc, sk, g, be):
    s = acc[0] + acc[1]
    o = s[:, :_D] / (s[:, _D:_D + 1] + 1e-16) + sk
    mu = jnp.mean(o, axis=1, keepdims=True)
    var = jnp.mean((o - mu) ** 2, axis=1, keepdims=True)
    return (o - mu) / jnp.sqrt(var + 1e-5) * g + be


def _comb_proj_body(acc_ref, sk_ref, g_ref, be_ref, w_ref, b_ref,
                    q_ref, kv_ref, sk2_ref):
    h = _norm_block(acc_ref[...], sk_ref[...], g_ref[...], be_ref[...])
    y = lax.dot_general(h, w_ref[...], (((1,), (1,)), ((), ())),
                        preferred_element_type=jnp.float32,
                        precision=lax.Precision.HIGHEST)
    y = y + b_ref[...]
    q_ref[...] = y[:, :_D]
    kv_ref[...] = y[:, _D:3 * _D]
    sk2_ref[...] = y[:, 3 * _D:]


def _comb_proj(acc, sk, g, be, w, b):
    return pl.pallas_call(
        _comb_proj_body,
        grid=(_GRID,),
        in_specs=[
            pl.BlockSpec((2, _ROWBLK, _AW), lambda i: (0, i, 0)),
            pl.BlockSpec((_ROWBLK, _D), lambda i: (i, 0)),
            pl.BlockSpec((1, _D), lambda i: (0, 0)),
            pl.BlockSpec((1, _D), lambda i: (0, 0)),
            pl.BlockSpec((4 * _D, _D), lambda i: (0, 0)),
            pl.BlockSpec((1, 4 * _D), lambda i: (0, 0)),
        ],
        out_specs=[
            pl.BlockSpec((_ROWBLK, _D), lambda i: (i, 0)),
            pl.BlockSpec((_ROWBLK, 2 * _D), lambda i: (i, 0)),
            pl.BlockSpec((_ROWBLK, _D), lambda i: (i, 0)),
        ],
        out_shape=[
            jax.ShapeDtypeStruct((_N, _D), jnp.float32),
            jax.ShapeDtypeStruct((_N, 2 * _D), jnp.float32),
            jax.ShapeDtypeStruct((_N, _D), jnp.float32),
        ],
    )(acc, sk, g, be, w, b)


def _comb_final_body(acc_ref, sk_ref, g_ref, be_ref, h_ref):
    h_ref[...] = _norm_block(acc_ref[...], sk_ref[...], g_ref[...], be_ref[...])


def _comb_final(acc, sk, g, be):
    return pl.pallas_call(
        _comb_final_body,
        grid=(_GRID,),
        in_specs=[
            pl.BlockSpec((2, _ROWBLK, _AW), lambda i: (0, i, 0)),
            pl.BlockSpec((_ROWBLK, _D), lambda i: (i, 0)),
            pl.BlockSpec((1, _D), lambda i: (0, 0)),
            pl.BlockSpec((1, _D), lambda i: (0, 0)),
        ],
        out_specs=pl.BlockSpec((_ROWBLK, _D), lambda i: (i, 0)),
        out_shape=jax.ShapeDtypeStruct((_N, _D), jnp.float32),
    )(acc, sk, g, be)


_INV_SQRT_C = 1.0 / math.sqrt(_D)


_CPS = 10
_NSEG = _NCHUNK // _CPS
_HB = _B // 2


@functools.partial(
    pl.kernel,
    mesh=plsc.VectorSubcoreMesh(core_axis_name="c", subcore_axis_name="s"),
    compiler_params=pltpu.CompilerParams(use_tc_tiling_on_sc=False,
                                         needs_layout_passes=False),
    out_type=jax.ShapeDtypeStruct((2, _N, _AW), jnp.float32),
    scratch_types=[
        pltpu.VMEM((2, _CPS, _B), jnp.int32),
        pltpu.VMEM((2, _CPS, _B), jnp.int32),
        pltpu.VMEM((2, _CPS, 2, _HB), jnp.int32),
        pltpu.VMEM((2, _B, _D), jnp.float32),
        pltpu.VMEM((2, _B, 2 * _D), jnp.float32),
        pltpu.VMEM((2, _HB, _AW), jnp.float32),
        pltpu.VMEM_SHARED((_N, _AW), jnp.float32),
        pltpu.SemaphoreType.DMA,
        pltpu.SemaphoreType.DMA,
        pltpu.SemaphoreType.DMA,
        pltpu.SemaphoreType.DMA,
    ],
)
def _edge_kernel(q_hbm, kv_hbm, src_hbm, dst_hbm, dsth_hbm, out_hbm,
                 didx, sidx, didxh, qr2, kvr2, stage, acc,
                 sem_q, sem_kv, sem_i, sem_s):
    cid = lax.axis_index("c")
    sid = lax.axis_index("s")

    def _zrow(i, carry):
        for hp in range(2):
            for c in range(_AW // 16):
                stage[hp, i, pl.ds(16 * c, 16)] = jnp.zeros((16,), jnp.float32)
        return carry
    lax.fori_loop(0, _HB, _zrow, 0)
    row0 = sid * _RPT
    nfull = _RPT // _HB
    rem = _RPT - nfull * _HB
    for r in range(nfull):
        pltpu.sync_copy(stage.at[0], acc.at[pl.ds(row0 + r * _HB, _HB)])
    pltpu.sync_copy(stage.at[0, pl.ds(0, rem)],
                    acc.at[pl.ds(row0 + nfull * _HB, rem)])
    plsc.subcore_barrier()

    cbase0 = (cid * 16 + sid) * (_EPT // _B)

    def _fire_gather(it):
        j = lax.rem(it, _CPS)
        sp = lax.rem(lax.div(it, _CPS), 2)
        p = lax.rem(it, 2)
        pltpu.async_copy(q_hbm.at[didx.at[sp, j]], qr2.at[p], sem_q)
        pltpu.async_copy(kv_hbm.at[sidx.at[sp, j]], kvr2.at[p], sem_kv)

    def _fire_ids(seg):
        sp = lax.rem(seg, 2)
        pltpu.async_copy(dst_hbm.at[pl.ds(cbase0 + seg * _CPS, _CPS)],
                         didx.at[sp], sem_i)
        pltpu.async_copy(src_hbm.at[pl.ds(cbase0 + seg * _CPS, _CPS)],
                         sidx.at[sp], sem_i)
        pltpu.async_copy(dsth_hbm.at[pl.ds(cbase0 + seg * _CPS, _CPS)],
                         didxh.at[sp], sem_i)

    def _wait_ids():
        pltpu.make_async_copy(dst_hbm.at[pl.ds(cbase0, _CPS)],
                              didx.at[0], sem_i).wait()
        pltpu.make_async_copy(src_hbm.at[pl.ds(cbase0, _CPS)],
                              sidx.at[0], sem_i).wait()
        pltpu.make_async_copy(dsth_hbm.at[pl.ds(cbase0, _CPS)],
                              didxh.at[0], sem_i).wait()

    _fire_ids(0)
    _wait_ids()
    _fire_gather(0)

    def _chunk(it, carry):
        j = lax.rem(it, _CPS)
        seg = lax.div(it, _CPS)
        sp = lax.rem(seg, 2)
        p = lax.rem(it, 2)

        pltpu.make_async_copy(q_hbm.at[didx.at[0, 0]], qr2.at[p],
                              sem_q).wait()
        pltpu.make_async_copy(kv_hbm.at[sidx.at[0, 0]], kvr2.at[p],
                              sem_kv).wait()

        @pl.when(jnp.logical_and(j == 0, seg < _NSEG - 1))
        def _():
            _fire_ids(seg + 1)

        @pl.when(jnp.logical_and(j == _CPS - 1, seg < _NSEG - 1))
        def _():
            _wait_ids()

        @pl.when(it < _NCHUNK - 1)
        def _():
            _fire_gather(it + 1)

        @pl.when(it > 0)
        def _():
            for _h in range(2):
                pltpu.make_async_copy(stage.at[0],
                                      acc.at[didxh.at[0, 0, 0]],
                                      sem_s).wait()

        for h in range(2):
            @plsc.parallel_loop(0, _HB, 1, unroll=4)
            def _edge(e, h=h):
                ge = h * _HB + e
                pr = qr2[p, ge, pl.ds(0, 16)] * kvr2[p, ge, pl.ds(0, 16)]
                for c in range(1, 8):
                    pr = pr + (qr2[p, ge, pl.ds(16 * c, 16)]
                               * kvr2[p, ge, pl.ds(16 * c, 16)])
                a = jnp.sum(pr) * _INV_SQRT_C
                s = jnp.exp(jnp.full((16,), a, jnp.float32))
                for c in range(8):
                    stage[h, e, pl.ds(16 * c, 16)] = (
                        s * kvr2[p, ge, pl.ds(_D + 16 * c, 16)])
                stage[h, e, pl.ds(_D, 16)] = s
            pltpu.async_copy(stage.at[h], acc.at[didxh.at[sp, j, h]],
                             sem_s, add=True)
        return carry

    lax.fori_loop(0, _NCHUNK, _chunk, 0)
    for _h in range(2):
        pltpu.make_async_copy(stage.at[0], acc.at[didxh.at[0, 0, 0]],
                              sem_s).wait()
    plsc.subcore_barrier()

    pltpu.sync_copy(acc.at[pl.ds(sid * _RPT, _RPT)],
                    out_hbm.at[cid, pl.ds(sid * _RPT, _RPT)])


def kernel(x, edge_index, Wq1, bq1, Wk1, bk1, Wv1, bv1, Ws1, bs1, g1, be1,
           Wq2, bq2, Wk2, bk2, Wv2, bv2, Ws2, bs2, g2, be2):
    w1 = jnp.concatenate([Wq1, Wk1, Wv1, Ws1], axis=0)
    b1 = jnp.concatenate([bq1, bk1, bv1, bs1])[None, :]
    w2 = jnp.concatenate([Wq2, Wk2, Wv2, Ws2], axis=0)
    b2 = jnp.concatenate([bq2, bk2, bv2, bs2])[None, :]
    src = edge_index[0].reshape(_E // _B, _B)
    dst = edge_index[1].reshape(_E // _B, _B)
    dsth = edge_index[1].reshape(_E // _B, 2, _HB)

    q1, kv1, sk1 = _proj(x, w1, b1)
    acc1 = _edge_kernel(q1, kv1, src, dst, dsth)
    q2, kv2, sk2 = _comb_proj(acc1, sk1, g1[None, :], be1[None, :], w2, b2)
    acc2 = _edge_kernel(q2, kv2, src, dst, dsth)
    return _comb_final(acc2, sk2, g2[None, :], be2[None, :])


# speedup vs baseline: 18.5402x; 1.0014x over previous
import functools
import math

import jax
import jax.numpy as jnp
from jax import lax
from jax.experimental import pallas as pl
from jax.experimental.pallas import tpu as pltpu
from jax.experimental.pallas import tpu_sc as plsc

_N = 10000
_E = 320000
_D = 128
_AW = _D + 16
_B = 40
_NTILES = 32
_EPT = _E // _NTILES
_NCHUNK = _EPT // _B
_RPT = _N // 16
_ROWBLK = 1000
_GRID = _N // _ROWBLK


def _proj_body(x_ref, w_ref, b_ref, q_ref, kv_ref, sk_ref):
    y = lax.dot_general(x_ref[...], w_ref[...], (((1,), (1,)), ((), ())),
                        preferred_element_type=jnp.float32,
                        precision=lax.Precision.HIGHEST)
    y = y + b_ref[...]
    q_ref[...] = y[:, :_D]
    kv_ref[...] = y[:, _D:3 * _D]
    sk_ref[...] = y[:, 3 * _D:]


def _proj(x, w, b):
    return pl.pallas_call(
        _proj_body,
        grid=(_GRID,),
        in_specs=[
            pl.BlockSpec((_ROWBLK, _D), lambda i: (i, 0)),
            pl.BlockSpec((4 * _D, _D), lambda i: (0, 0)),
            pl.BlockSpec((1, 4 * _D), lambda i: (0, 0)),
        ],
        out_specs=[
            pl.BlockSpec((_ROWBLK, _D), lambda i: (i, 0)),
            pl.BlockSpec((_ROWBLK, 2 * _D), lambda i: (i, 0)),
            pl.BlockSpec((_ROWBLK, _D), lambda i: (i, 0)),
        ],
        out_shape=[
            jax.ShapeDtypeStruct((_N, _D), jnp.float32),
            jax.ShapeDtypeStruct((_N, 2 * _D), jnp.float32),
            jax.ShapeDtypeStruct((_N, _D), jnp.float32),
        ],
    )(x, w, b)


def _norm_block(acc, sk, g, be):
    s = acc[0] + acc[1]
    o = s[:, :_D] / (s[:, _D:_D + 1] + 1e-16) + sk
    mu = jnp.mean(o, axis=1, keepdims=True)
    var = jnp.mean((o - mu) ** 2, axis=1, keepdims=True)
    return (o - mu) / jnp.sqrt(var + 1e-5) * g + be


def _comb_proj_body(acc_ref, sk_ref, g_ref, be_ref, w_ref, b_ref,
                    q_ref, kv_ref, sk2_ref):
    h = _norm_block(acc_ref[...], sk_ref[...], g_ref[...], be_ref[...])
    y = lax.dot_general(h, w_ref[...], (((1,), (1,)), ((), ())),
                        preferred_element_type=jnp.float32,
                        precision=lax.Precision.HIGHEST)
    y = y + b_ref[...]
    q_ref[...] = y[:, :_D]
    kv_ref[...] = y[:, _D:3 * _D]
    sk2_ref[...] = y[:, 3 * _D:]


def _comb_proj(acc, sk, g, be, w, b):
    return pl.pallas_call(
        _comb_proj_body,
        grid=(_GRID,),
        in_specs=[
            pl.BlockSpec((2, _ROWBLK, _AW), lambda i: (0, i, 0)),
            pl.BlockSpec((_ROWBLK, _D), lambda i: (i, 0)),
            pl.BlockSpec((1, _D), lambda i: (0, 0)),
            pl.BlockSpec((1, _D), lambda i: (0, 0)),
            pl.BlockSpec((4 * _D, _D), lambda i: (0, 0)),
            pl.BlockSpec((1, 4 * _D), lambda i: (0, 0)),
        ],
        out_specs=[
            pl.BlockSpec((_ROWBLK, _D), lambda i: (i, 0)),
            pl.BlockSpec((_ROWBLK, 2 * _D), lambda i: (i, 0)),
            pl.BlockSpec((_ROWBLK, _D), lambda i: (i, 0)),
        ],
        out_shape=[
            jax.ShapeDtypeStruct((_N, _D), jnp.float32),
            jax.ShapeDtypeStruct((_N, 2 * _D), jnp.float32),
            jax.ShapeDtypeStruct((_N, _D), jnp.float32),
        ],
    )(acc, sk, g, be, w, b)


def _comb_final_body(acc_ref, sk_ref, g_ref, be_ref, h_ref):
    h_ref[...] = _norm_block(acc_ref[...], sk_ref[...], g_ref[...], be_ref[...])


def _comb_final(acc, sk, g, be):
    return pl.pallas_call(
        _comb_final_body,
        grid=(_GRID,),
        in_specs=[
            pl.BlockSpec((2, _ROWBLK, _AW), lambda i: (0, i, 0)),
            pl.BlockSpec((_ROWBLK, _D), lambda i: (i, 0)),
            pl.BlockSpec((1, _D), lambda i: (0, 0)),
            pl.BlockSpec((1, _D), lambda i: (0, 0)),
        ],
        out_specs=pl.BlockSpec((_ROWBLK, _D), lambda i: (i, 0)),
        out_shape=jax.ShapeDtypeStruct((_N, _D), jnp.float32),
    )(acc, sk, g, be)


_INV_SQRT_C = 1.0 / math.sqrt(_D)


_CPS = 10
_NSEG = _NCHUNK // _CPS
_HB = _B // 2


@functools.partial(
    pl.kernel,
    mesh=plsc.VectorSubcoreMesh(core_axis_name="c", subcore_axis_name="s"),
    compiler_params=pltpu.CompilerParams(use_tc_tiling_on_sc=False,
                                         needs_layout_passes=False),
    out_type=jax.ShapeDtypeStruct((2, _N, _AW), jnp.float32),
    scratch_types=[
        pltpu.VMEM((2, _CPS, _B), jnp.int32),
        pltpu.VMEM((2, _CPS, _B), jnp.int32),
        pltpu.VMEM((2, _CPS, 2, _HB), jnp.int32),
        pltpu.VMEM((2, _B, _D), jnp.float32),
        pltpu.VMEM((2, _B, 2 * _D), jnp.float32),
        pltpu.VMEM((2, _HB, _AW), jnp.float32),
        pltpu.VMEM_SHARED((_N, _AW), jnp.float32),
        pltpu.SemaphoreType.DMA,
        pltpu.SemaphoreType.DMA,
        pltpu.SemaphoreType.DMA,
        pltpu.SemaphoreType.DMA,
    ],
)
def _edge_kernel(q_hbm, kv_hbm, src_hbm, dst_hbm, dsth_hbm, out_hbm,
                 didx, sidx, didxh, qr2, kvr2, stage, acc,
                 sem_q, sem_kv, sem_i, sem_s):
    cid = lax.axis_index("c")
    sid = lax.axis_index("s")

    def _zrow(i, carry):
        for hp in range(2):
            for c in range(_AW // 16):
                stage[hp, i, pl.ds(16 * c, 16)] = jnp.zeros((16,), jnp.float32)
        return carry
    lax.fori_loop(0, _HB, _zrow, 0)
    row0 = sid * _RPT
    nfull = _RPT // _HB
    rem = _RPT - nfull * _HB
    for r in range(nfull):
        pltpu.sync_copy(stage.at[0], acc.at[pl.ds(row0 + r * _HB, _HB)])
    pltpu.sync_copy(stage.at[0, pl.ds(0, rem)],
                    acc.at[pl.ds(row0 + nfull * _HB, rem)])
    plsc.subcore_barrier()

    cbase0 = (cid * 16 + sid) * (_EPT // _B)

    def _fire_gather(it):
        j = lax.rem(it, _CPS)
        sp = lax.rem(lax.div(it, _CPS), 2)
        p = lax.rem(it, 2)
        pltpu.async_copy(q_hbm.at[didx.at[sp, j]], qr2.at[p], sem_q)
        pltpu.async_copy(kv_hbm.at[sidx.at[sp, j]], kvr2.at[p], sem_kv)

    def _fire_ids(seg):
        sp = lax.rem(seg, 2)
        pltpu.async_copy(dst_hbm.at[pl.ds(cbase0 + seg * _CPS, _CPS)],
                         didx.at[sp], sem_i)
        pltpu.async_copy(src_hbm.at[pl.ds(cbase0 + seg * _CPS, _CPS)],
                         sidx.at[sp], sem_i)
        pltpu.async_copy(dsth_hbm.at[pl.ds(cbase0 + seg * _CPS, _CPS)],
                         didxh.at[sp], sem_i)

    def _wait_ids():
        pltpu.make_async_copy(dst_hbm.at[pl.ds(cbase0, _CPS)],
                              didx.at[0], sem_i).wait()
        pltpu.make_async_copy(src_hbm.at[pl.ds(cbase0, _CPS)],
                              sidx.at[0], sem_i).wait()
        pltpu.make_async_copy(dsth_hbm.at[pl.ds(cbase0, _CPS)],
                              didxh.at[0], sem_i).wait()

    _fire_ids(0)
    _wait_ids()
    _fire_gather(0)

    def _chunk(it, carry):
        j = lax.rem(it, _CPS)
        seg = lax.div(it, _CPS)
        sp = lax.rem(seg, 2)
        p = lax.rem(it, 2)

        pltpu.make_async_copy(q_hbm.at[didx.at[0, 0]], qr2.at[p],
                              sem_q).wait()
        pltpu.make_async_copy(kv_hbm.at[sidx.at[0, 0]], kvr2.at[p],
                              sem_kv).wait()

        @pl.when(jnp.logical_and(j == 0, seg < _NSEG - 1))
        def _():
            _fire_ids(seg + 1)

        @pl.when(jnp.logical_and(j == _CPS - 1, seg < _NSEG - 1))
        def _():
            _wait_ids()

        @pl.when(it < _NCHUNK - 1)
        def _():
            _fire_gather(it + 1)

        @pl.when(it > 0)
        def _():
            for _h in range(2):
                pltpu.make_async_copy(stage.at[0],
                                      acc.at[didxh.at[0, 0, 0]],
                                      sem_s).wait()

        for h in range(2):
            @plsc.parallel_loop(0, _HB, 1, unroll=5)
            def _edge(e, h=h):
                ge = h * _HB + e
                pr = qr2[p, ge, pl.ds(0, 16)] * kvr2[p, ge, pl.ds(0, 16)]
                for c in range(1, 8):
                    pr = pr + (qr2[p, ge, pl.ds(16 * c, 16)]
                               * kvr2[p, ge, pl.ds(16 * c, 16)])
                a = jnp.sum(pr) * _INV_SQRT_C
                s = jnp.exp(jnp.full((16,), a, jnp.float32))
                for c in range(8):
                    stage[h, e, pl.ds(16 * c, 16)] = (
                        s * kvr2[p, ge, pl.ds(_D + 16 * c, 16)])
                stage[h, e, pl.ds(_D, 16)] = s
            pltpu.async_copy(stage.at[h], acc.at[didxh.at[sp, j, h]],
                             sem_s, add=True)
        return carry

    lax.fori_loop(0, _NCHUNK, _chunk, 0)
    for _h in range(2):
        pltpu.make_async_copy(stage.at[0], acc.at[didxh.at[0, 0, 0]],
                              sem_s).wait()
    plsc.subcore_barrier()

    pltpu.sync_copy(acc.at[pl.ds(sid * _RPT, _RPT)],
                    out_hbm.at[cid, pl.ds(sid * _RPT, _RPT)])


def kernel(x, edge_index, Wq1, bq1, Wk1, bk1, Wv1, bv1, Ws1, bs1, g1, be1,
           Wq2, bq2, Wk2, bk2, Wv2, bv2, Ws2, bs2, g2, be2):
    w1 = jnp.concatenate([Wq1, Wk1, Wv1, Ws1], axis=0)
    b1 = jnp.concatenate([bq1, bk1, bv1, bs1])[None, :]
    w2 = jnp.concatenate([Wq2, Wk2, Wv2, Ws2], axis=0)
    b2 = jnp.concatenate([bq2, bk2, bv2, bs2])[None, :]
    src = edge_index[0].reshape(_E // _B, _B)
    dst = edge_index[1].reshape(_E // _B, _B)
    dsth = edge_index[1].reshape(_E // _B, 2, _HB)

    q1, kv1, sk1 = _proj(x, w1, b1)
    acc1 = _edge_kernel(q1, kv1, src, dst, dsth)
    q2, kv2, sk2 = _comb_proj(acc1, sk1, g1[None, :], be1[None, :], w2, b2)
    acc2 = _edge_kernel(q2, kv2, src, dst, dsth)
    return _comb_final(acc2, sk2, g2[None, :], be2[None, :])


# speedup vs baseline: 22.4638x; 1.2116x over previous
import functools
import math

import jax
import jax.numpy as jnp
import numpy as np
from jax import lax
from jax.experimental import pallas as pl
from jax.experimental.pallas import tpu as pltpu
from jax.experimental.pallas import tpu_sc as plsc

_N = 10000
_E = 320000
_D = 128
_AW = _D + 16
_B = 40
_NTILES = 32
_EPT = _E // _NTILES
_NCHUNK = _EPT // _B
_RPT = _N // 16
_ROWBLK = 2000
_GRID = _N // _ROWBLK


def _proj_body(x_ref, w_ref, b_ref, q_ref, kv_ref, sk_ref):
    y = lax.dot_general(x_ref[...], w_ref[...], (((1,), (1,)), ((), ())),
                        preferred_element_type=jnp.float32,
                        precision=lax.Precision.HIGHEST)
    y = y + b_ref[...]
    q_ref[...] = y[:, :_D].astype(jnp.bfloat16)
    kv_ref[...] = y[:, _D:3 * _D].astype(jnp.bfloat16)
    sk_ref[...] = y[:, 3 * _D:]


def _proj(x, w, b):
    return pl.pallas_call(
        _proj_body,
        grid=(_GRID,),
        in_specs=[
            pl.BlockSpec((_ROWBLK, _D), lambda i: (i, 0)),
            pl.BlockSpec((4 * _D, _D), lambda i: (0, 0)),
            pl.BlockSpec((1, 4 * _D), lambda i: (0, 0)),
        ],
        out_specs=[
            pl.BlockSpec((_ROWBLK, _D), lambda i: (i, 0)),
            pl.BlockSpec((_ROWBLK, 2 * _D), lambda i: (i, 0)),
            pl.BlockSpec((_ROWBLK, _D), lambda i: (i, 0)),
        ],
        out_shape=[
            jax.ShapeDtypeStruct((_N, _D), jnp.bfloat16),
            jax.ShapeDtypeStruct((_N, 2 * _D), jnp.bfloat16),
            jax.ShapeDtypeStruct((_N, _D), jnp.float32),
        ],
    )(x, w, b)


def _norm_block(acc, sk, g, be):
    s = acc[0] + acc[1]
    o = s[:, :_D] / (s[:, _D:_D + 1] + 1e-16) + sk
    mu = jnp.mean(o, axis=1, keepdims=True)
    var = jnp.mean((o - mu) ** 2, axis=1, keepdims=True)
    return (o - mu) / jnp.sqrt(var + 1e-5) * g + be


def _comb_proj_body(acc_ref, sk_ref, g_ref, be_ref, w_ref, b_ref,
                    q_ref, kv_ref, sk2_ref):
    h = _norm_block(acc_ref[...], sk_ref[...], g_ref[...], be_ref[...])
    y = lax.dot_general(h, w_ref[...], (((1,), (1,)), ((), ())),
                        preferred_element_type=jnp.float32,
                        precision=lax.Precision.HIGHEST)
    y = y + b_ref[...]
    q_ref[...] = y[:, :_D].astype(jnp.bfloat16)
    kv_ref[...] = y[:, _D:3 * _D].astype(jnp.bfloat16)
    sk2_ref[...] = y[:, 3 * _D:]


def _comb_proj(acc, sk, g, be, w, b):
    return pl.pallas_call(
        _comb_proj_body,
        grid=(_GRID,),
        in_specs=[
            pl.BlockSpec((2, _ROWBLK, _AW), lambda i: (0, i, 0)),
            pl.BlockSpec((_ROWBLK, _D), lambda i: (i, 0)),
            pl.BlockSpec((1, _D), lambda i: (0, 0)),
            pl.BlockSpec((1, _D), lambda i: (0, 0)),
            pl.BlockSpec((4 * _D, _D), lambda i: (0, 0)),
            pl.BlockSpec((1, 4 * _D), lambda i: (0, 0)),
        ],
        out_specs=[
            pl.BlockSpec((_ROWBLK, _D), lambda i: (i, 0)),
            pl.BlockSpec((_ROWBLK, 2 * _D), lambda i: (i, 0)),
            pl.BlockSpec((_ROWBLK, _D), lambda i: (i, 0)),
        ],
        out_shape=[
            jax.ShapeDtypeStruct((_N, _D), jnp.bfloat16),
            jax.ShapeDtypeStruct((_N, 2 * _D), jnp.bfloat16),
            jax.ShapeDtypeStruct((_N, _D), jnp.float32),
        ],
    )(acc, sk, g, be, w, b)


def _comb_final_body(acc_ref, sk_ref, g_ref, be_ref, h_ref):
    h_ref[...] = _norm_block(acc_ref[...], sk_ref[...], g_ref[...], be_ref[...])


def _comb_final(acc, sk, g, be):
    return pl.pallas_call(
        _comb_final_body,
        grid=(_GRID,),
        in_specs=[
            pl.BlockSpec((2, _ROWBLK, _AW), lambda i: (0, i, 0)),
            pl.BlockSpec((_ROWBLK, _D), lambda i: (i, 0)),
            pl.BlockSpec((1, _D), lambda i: (0, 0)),
            pl.BlockSpec((1, _D), lambda i: (0, 0)),
        ],
        out_specs=pl.BlockSpec((_ROWBLK, _D), lambda i: (i, 0)),
        out_shape=jax.ShapeDtypeStruct((_N, _D), jnp.float32),
    )(acc, sk, g, be)


_INV_SQRT_C = 1.0 / math.sqrt(_D)


_CPS = 10
_NSEG = _NCHUNK // _CPS
_HB = _B // 2


@functools.partial(
    pl.kernel,
    mesh=plsc.VectorSubcoreMesh(core_axis_name="c", subcore_axis_name="s"),
    compiler_params=pltpu.CompilerParams(use_tc_tiling_on_sc=False,
                                         needs_layout_passes=False),
    out_type=jax.ShapeDtypeStruct((2, _N, _AW), jnp.float32),
    scratch_types=[
        pltpu.VMEM((2, _CPS, _B), jnp.int32),
        pltpu.VMEM((2, _CPS, _B), jnp.int32),
        pltpu.VMEM((2, _CPS, 2, _HB), jnp.int32),
        pltpu.VMEM((2, _B, _D), jnp.bfloat16),
        pltpu.VMEM((2, _B, 2 * _D), jnp.bfloat16),
        pltpu.VMEM((2, _HB, _AW), jnp.float32),
        pltpu.VMEM_SHARED((_N, _AW), jnp.float32),
        pltpu.SemaphoreType.DMA,
        pltpu.SemaphoreType.DMA,
        pltpu.SemaphoreType.DMA,
        pltpu.SemaphoreType.DMA,
    ],
)
def _edge_kernel(q_hbm, kv_hbm, src_hbm, dst_hbm, dsth_hbm, out_hbm,
                 didx, sidx, didxh, qr2, kvr2, stage, acc,
                 sem_q, sem_kv, sem_i, sem_s):
    cid = lax.axis_index("c")
    sid = lax.axis_index("s")

    def _zrow(i, carry):
        for hp in range(2):
            for c in range(_AW // 16):
                stage[hp, i, pl.ds(16 * c, 16)] = jnp.zeros((16,), jnp.float32)
        return carry
    lax.fori_loop(0, _HB, _zrow, 0)
    row0 = sid * _RPT
    nfull = _RPT // _HB
    rem = _RPT - nfull * _HB
    for r in range(nfull):
        pltpu.sync_copy(stage.at[0], acc.at[pl.ds(row0 + r * _HB, _HB)])
    pltpu.sync_copy(stage.at[0, pl.ds(0, rem)],
                    acc.at[pl.ds(row0 + nfull * _HB, rem)])
    plsc.subcore_barrier()

    cbase0 = (cid * 16 + sid) * (_EPT // _B)

    def _fire_gather(it):
        j = lax.rem(it, _CPS)
        sp = lax.rem(lax.div(it, _CPS), 2)
        p = lax.rem(it, 2)
        pltpu.async_copy(q_hbm.at[didx.at[sp, j]], qr2.at[p], sem_q)
        pltpu.async_copy(kv_hbm.at[sidx.at[sp, j]], kvr2.at[p], sem_kv)

    def _fire_ids(seg):
        sp = lax.rem(seg, 2)
        pltpu.async_copy(dst_hbm.at[pl.ds(cbase0 + seg * _CPS, _CPS)],
                         didx.at[sp], sem_i)
        pltpu.async_copy(src_hbm.at[pl.ds(cbase0 + seg * _CPS, _CPS)],
                         sidx.at[sp], sem_i)
        pltpu.async_copy(dsth_hbm.at[pl.ds(cbase0 + seg * _CPS, _CPS)],
                         didxh.at[sp], sem_i)

    def _wait_ids():
        pltpu.make_async_copy(dst_hbm.at[pl.ds(cbase0, _CPS)],
                              didx.at[0], sem_i).wait()
        pltpu.make_async_copy(src_hbm.at[pl.ds(cbase0, _CPS)],
                              sidx.at[0], sem_i).wait()
        pltpu.make_async_copy(dsth_hbm.at[pl.ds(cbase0, _CPS)],
                              didxh.at[0], sem_i).wait()

    _fire_ids(0)
    _wait_ids()
    _fire_gather(0)

    def _chunk(it, carry):
        j = lax.rem(it, _CPS)
        seg = lax.div(it, _CPS)
        sp = lax.rem(seg, 2)
        p = lax.rem(it, 2)

        pltpu.make_async_copy(q_hbm.at[didx.at[0, 0]], qr2.at[p],
                              sem_q).wait()
        pltpu.make_async_copy(kv_hbm.at[sidx.at[0, 0]], kvr2.at[p],
                              sem_kv).wait()

        @pl.when(jnp.logical_and(j == 0, seg < _NSEG - 1))
        def _():
            _fire_ids(seg + 1)

        @pl.when(jnp.logical_and(j == _CPS - 1, seg < _NSEG - 1))
        def _():
            _wait_ids()

        @pl.when(it < _NCHUNK - 1)
        def _():
            _fire_gather(it + 1)

        @pl.when(it > 0)
        def _():
            for _h in range(2):
                pltpu.make_async_copy(stage.at[0],
                                      acc.at[didxh.at[0, 0, 0]],
                                      sem_s).wait()

        for h in range(2):
            @plsc.parallel_loop(0, _HB, 1, unroll=5)
            def _edge(e, h=h):
                ge = h * _HB + e
                pr = None
                for c in range(4):
                    qa, qb = plsc.unpack(
                        qr2[p, ge, pl.ds(32 * c, 32)],
                        format=plsc.PackFormat.INTERLEAVED,
                        preferred_element_type=jnp.float32)
                    ka, kb = plsc.unpack(
                        kvr2[p, ge, pl.ds(32 * c, 32)],
                        format=plsc.PackFormat.INTERLEAVED,
                        preferred_element_type=jnp.float32)
                    t = qa * ka + qb * kb
                    pr = t if pr is None else pr + t
                a = jnp.sum(pr) * _INV_SQRT_C
                s = jnp.exp(jnp.full((16,), a, jnp.float32))
                for c in range(4):
                    va, vb = plsc.unpack(
                        kvr2[p, ge, pl.ds(_D + 32 * c, 32)],
                        format=plsc.PackFormat.INTERLEAVED,
                        preferred_element_type=jnp.float32)
                    stage[h, e, pl.ds(32 * c, 16)] = s * va
                    stage[h, e, pl.ds(32 * c + 16, 16)] = s * vb
                stage[h, e, pl.ds(_D, 16)] = s
            pltpu.async_copy(stage.at[h], acc.at[didxh.at[sp, j, h]],
                             sem_s, add=True)
        return carry

    lax.fori_loop(0, _NCHUNK, _chunk, 0)
    for _h in range(2):
        pltpu.make_async_copy(stage.at[0], acc.at[didxh.at[0, 0, 0]],
                              sem_s).wait()
    plsc.subcore_barrier()

    pltpu.sync_copy(acc.at[pl.ds(sid * _RPT, _RPT)],
                    out_hbm.at[cid, pl.ds(sid * _RPT, _RPT)])


_SIGMA = np.concatenate(
    [np.concatenate([np.arange(b * 32, b * 32 + 32, 2),
                     np.arange(b * 32 + 1, b * 32 + 32, 2)])
     for b in range(4)])
_VPERM = np.argsort(_SIGMA)


def kernel(x, edge_index, Wq1, bq1, Wk1, bk1, Wv1, bv1, Ws1, bs1, g1, be1,
           Wq2, bq2, Wk2, bk2, Wv2, bv2, Ws2, bs2, g2, be2):
    w1 = jnp.concatenate([Wq1, Wk1, Wv1[_VPERM], Ws1], axis=0)
    b1 = jnp.concatenate([bq1, bk1, bv1[_VPERM], bs1])[None, :]
    w2 = jnp.concatenate([Wq2, Wk2, Wv2[_VPERM], Ws2], axis=0)
    b2 = jnp.concatenate([bq2, bk2, bv2[_VPERM], bs2])[None, :]
    src = edge_index[0].reshape(_E // _B, _B)
    dst = edge_index[1].reshape(_E // _B, _B)
    dsth = edge_index[1].reshape(_E // _B, 2, _HB)

    q1, kv1, sk1 = _proj(x, w1, b1)
    acc1 = _edge_kernel(q1, kv1, src, dst, dsth)
    q2, kv2, sk2 = _comb_proj(acc1, sk1, g1[None, :], be1[None, :], w2, b2)
    acc2 = _edge_kernel(q2, kv2, src, dst, dsth)
    return _comb_final(acc2, sk2, g2[None, :], be2[None, :])
